# baseline (speedup 1.0000x reference)
import jax
import jax.numpy as jnp
from jax.experimental import pallas as pl
from jax.experimental.pallas import tpu as pltpu


def _mlp_kernel(x_ref,
                w1_ref, b1_ref,
                w2_ref, b2_ref,
                w3_ref, b3_ref,
                w4_ref, b4_ref,
                o_ref):
    x = x_ref[...]

    h = jnp.dot(x, w1_ref[...], preferred_element_type=jnp.float32)
    h = jnp.tanh(h + b1_ref[...])

    h = jnp.dot(h, w2_ref[...], preferred_element_type=jnp.float32)
    h = jnp.tanh(h + b2_ref[...])

    h = jnp.dot(h, w3_ref[...], preferred_element_type=jnp.float32)
    h = jnp.tanh(h + b3_ref[...])

    y = jnp.dot(h, w4_ref[...], preferred_element_type=jnp.float32)
    o_ref[...] = y + b4_ref[...]


def _round_up(n, m):
    return ((n + m - 1) // m) * m


def kernel(x, w1, b1, w2, b2, w3, b3, w4, b4):
    B = x.shape[0]
    if x.ndim != 2 or x.dtype != jnp.float32:
        x2d = x.reshape(B, -1).astype(jnp.float32)
    else:
        x2d = x
    f_in = x2d.shape[1]
    n_classes = w4.shape[0]

    w1t = w1.T.astype(jnp.float32)
    w2t = w2.T.astype(jnp.float32)
    w3t = w3.T.astype(jnp.float32)
    w4t = w4.T.astype(jnp.float32)
    b1r = b1.reshape(1, -1).astype(jnp.float32)
    b2r = b2.reshape(1, -1).astype(jnp.float32)
    b3r = b3.reshape(1, -1).astype(jnp.float32)
    b4r = b4.reshape(1, -1).astype(jnp.float32)

    TB = min(8192, _round_up(B, 8))
    B_pad = _round_up(B, TB)
    if B_pad != B:
        x2d = jnp.pad(x2d, ((0, B_pad - B), (0, 0)))
    n_tiles = B_pad // TB

    resident = lambda shape: pl.BlockSpec(shape, lambda i: (0, 0))

    y = pl.pallas_call(
        _mlp_kernel,
        out_shape=jax.ShapeDtypeStruct((B_pad, n_classes), jnp.float32),
        grid=(n_tiles,),
        in_specs=[
            pl.BlockSpec((TB, f_in), lambda i: (i, 0)),
            resident(w1t.shape), resident(b1r.shape),
            resident(w2t.shape), resident(b2r.shape),
            resident(w3t.shape), resident(b3r.shape),
            resident(w4t.shape), resident(b4r.shape),
        ],
        out_specs=pl.BlockSpec((TB, n_classes), lambda i: (i, 0)),
        compiler_params=pltpu.CompilerParams(
            dimension_semantics=("parallel",)),
    )(x2d, w1t, b1r, w2t, b2r, w3t, b3r, w4t, b4r)

    return y[:B]


# speedup vs baseline: 3.6557x; 3.6557x over previous
import jax
import jax.numpy as jnp
from jax.experimental import pallas as pl
from jax.experimental.pallas import tpu as pltpu


def _mlp_kernel(x_ref,
                w1_ref, b1_ref,
                w2_ref, b2_ref,
                w3_ref, b3_ref,
                w4_ref, b4_ref,
                o_ref):
    x = x_ref[...]

    h = jnp.dot(w1_ref[...], x, preferred_element_type=jnp.float32)
    h = jnp.tanh(h + b1_ref[...])

    h = jnp.dot(w2_ref[...], h, preferred_element_type=jnp.float32)
    h = jnp.tanh(h + b2_ref[...])

    h = jnp.dot(w3_ref[...], h, preferred_element_type=jnp.float32)
    h = jnp.tanh(h + b3_ref[...])

    y = jnp.dot(w4_ref[...], h, preferred_element_type=jnp.float32)
    o_ref[...] = y + b4_ref[...]


def _round_up(n, m):
    return ((n + m - 1) // m) * m


def kernel(x, w1, b1, w2, b2, w3, b3, w4, b4):
    B = x.shape[0]
    x2d = x.reshape(B, -1)
    if x2d.dtype != jnp.float32:
        x2d = x2d.astype(jnp.float32)
    f_in = x2d.shape[1]
    n_classes = w4.shape[0]

    xt = x2d.T

    w1f = w1.astype(jnp.float32)
    w2f = w2.astype(jnp.float32)
    w3f = w3.astype(jnp.float32)
    w4f = w4.astype(jnp.float32)
    b1c = b1.reshape(-1, 1).astype(jnp.float32)
    b2c = b2.reshape(-1, 1).astype(jnp.float32)
    b3c = b3.reshape(-1, 1).astype(jnp.float32)
    b4c = b4.reshape(-1, 1).astype(jnp.float32)

    TBL = min(8192, _round_up(B, 128))
    B_pad = _round_up(B, TBL)
    if B_pad != B:
        xt = jnp.pad(xt, ((0, 0), (0, B_pad - B)))
    n_tiles = B_pad // TBL

    resident = lambda shape: pl.BlockSpec(shape, lambda i: (0, 0))

    yt = pl.pallas_call(
        _mlp_kernel,
        out_shape=jax.ShapeDtypeStruct((n_classes, B_pad), jnp.float32),
        grid=(n_tiles,),
        in_specs=[
            pl.BlockSpec((f_in, TBL), lambda i: (0, i)),
            resident(w1f.shape), resident(b1c.shape),
            resident(w2f.shape), resident(b2c.shape),
            resident(w3f.shape), resident(b3c.shape),
            resident(w4f.shape), resident(b4c.shape),
        ],
        out_specs=pl.BlockSpec((n_classes, TBL), lambda i: (0, i)),
        compiler_params=pltpu.CompilerParams(
            dimension_semantics=("parallel",)),
    )(xt, w1f, b1c, w2f, b2c, w3f, b3c, w4f, b4c)

    return yt[:, :B].T
